# baseline (speedup 1.0000x reference)
import jax
import jax.numpy as jnp
from jax.experimental import pallas as pl
from jax.experimental.pallas import tpu as pltpu

T, N, D, H, L = 512, 16, 512, 512, 2
TT = 64
GRID = T // TT
TPAD = T + 8


def _lstm_chunk_kernel(x_ref, mask_ref, hc0_ref, wih0_ref, whh0_ref,
                       wih1_ref, whh1_ref, b0_ref, b1_ref,
                       ys_ref, fin_ref, g0_ref, ma_ref, mb_ref):
    i = pl.program_id(0)
    base = i * TT

    def mrow(idx):
        return mask_ref[pl.ds(idx, 1)].reshape(N, 128)[:, :1]

    @pl.when(i == 0)
    def _():
        fin_ref[...] = hc0_ref[...]
        m0 = mrow(0)
        h0m0 = (hc0_ref[0] * m0).astype(jnp.bfloat16)
        h1m0 = (hc0_ref[1] * m0).astype(jnp.bfloat16)
        ma_ref[...] = jnp.dot(h0m0, whh0_ref[...],
                              preferred_element_type=jnp.float32)
        mb_ref[...] = jnp.dot(h1m0, whh1_ref[...],
                              preferred_element_type=jnp.float32) + b1_ref[...]

    xv = x_ref[...].reshape(TT * N, D).astype(jnp.bfloat16)
    g0_ref[...] = (
        jnp.dot(xv, wih0_ref[...], preferred_element_type=jnp.float32)
        + b0_ref[...]
    )

    init = (fin_ref[0], fin_ref[2], fin_ref[1], fin_ref[3],
            ma_ref[...], mb_ref[...], mrow(base))

    def step(t, carry):
        h0, c0, h1, c1, ma, mb, mt = carry
        mt1 = mrow(base + t + 1)

        c0m = c0 * mt
        g0 = g0_ref[pl.ds(t * N, N), :] + ma
        i0 = jax.nn.sigmoid(g0[:, :H])
        f0 = jax.nn.sigmoid(g0[:, H:2 * H])
        gg0 = jnp.tanh(g0[:, 2 * H:3 * H])
        o0 = jax.nn.sigmoid(g0[:, 3 * H:])
        c0n = f0 * c0m + i0 * gg0
        h0n = o0 * jnp.tanh(c0n)
        h0nb = h0n.astype(jnp.bfloat16)

        g1 = jnp.dot(h0nb, wih1_ref[...],
                     preferred_element_type=jnp.float32) + mb

        h0m_next = (h0n * mt1).astype(jnp.bfloat16)
        ma_next = jnp.dot(h0m_next, whh0_ref[...],
                          preferred_element_type=jnp.float32)

        c1m = c1 * mt
        i1 = jax.nn.sigmoid(g1[:, :H])
        f1 = jax.nn.sigmoid(g1[:, H:2 * H])
        gg1 = jnp.tanh(g1[:, 2 * H:3 * H])
        o1 = jax.nn.sigmoid(g1[:, 3 * H:])
        c1n = f1 * c1m + i1 * gg1
        h1n = o1 * jnp.tanh(c1n)

        h1m_next = (h1n * mt1).astype(jnp.bfloat16)
        mb_next = jnp.dot(h1m_next, whh1_ref[...],
                          preferred_element_type=jnp.float32) + b1_ref[...]

        ys_ref[pl.ds(t, 1)] = h1n[None]
        return (h0n, c0n, h1n, c1n, ma_next, mb_next, mt1)

    h0, c0, h1, c1, ma, mb, _ = jax.lax.fori_loop(0, TT, step, init)
    fin_ref[0] = h0
    fin_ref[1] = h1
    fin_ref[2] = c0
    fin_ref[3] = c1
    ma_ref[...] = ma
    mb_ref[...] = mb


def kernel(x, rnn_states, dones, W_ih0, W_hh0, b_ih0, b_hh0,
           W_ih1, W_hh1, b_ih1, b_hh1):
    xs = x.reshape(T, N, D)
    mask_b = jnp.zeros((TPAD, N, 128), jnp.float32)
    mask_b = mask_b.at[:T].set(
        jnp.broadcast_to((1.0 - dones.astype(jnp.float32))[:, :, None],
                         (T, N, 128)))
    wih0 = W_ih0.T.astype(jnp.bfloat16)
    whh0 = W_hh0.T.astype(jnp.bfloat16)
    wih1 = W_ih1.T.astype(jnp.bfloat16)
    whh1 = W_hh1.T.astype(jnp.bfloat16)
    b0 = (b_ih0 + b_hh0).reshape(1, 4 * H)
    b1 = (b_ih1 + b_hh1).reshape(1, 4 * H)

    full = lambda shape: pl.BlockSpec(shape, lambda i: (0,) * len(shape))

    ys, fin = pl.pallas_call(
        _lstm_chunk_kernel,
        grid=(GRID,),
        in_specs=[
            pl.BlockSpec((TT, N, D), lambda i: (i, 0, 0)),
            full((TPAD, N, 128)),
            full((2 * L, N, H)),
            full((D, 4 * H)),
            full((H, 4 * H)),
            full((H, 4 * H)),
            full((H, 4 * H)),
            full((1, 4 * H)),
            full((1, 4 * H)),
        ],
        out_specs=[
            pl.BlockSpec((TT, N, H), lambda i: (i, 0, 0)),
            full((2 * L, N, H)),
        ],
        out_shape=[
            jax.ShapeDtypeStruct((T, N, H), jnp.float32),
            jax.ShapeDtypeStruct((2 * L, N, H), jnp.float32),
        ],
        scratch_shapes=[
            pltpu.VMEM((TT * N, 4 * H), jnp.float32),
            pltpu.VMEM((N, 4 * H), jnp.float32),
            pltpu.VMEM((N, 4 * H), jnp.float32),
        ],
    )(xs, mask_b, rnn_states, wih0, whh0, wih1, whh1, b0, b1)

    return ys.reshape(T * N, H), fin


# speedup vs baseline: 1.0122x; 1.0122x over previous
import jax
import jax.numpy as jnp
from jax.experimental import pallas as pl
from jax.experimental.pallas import tpu as pltpu

T, N, D, H, L = 512, 16, 512, 512, 2
TT = 64
GRID = T // TT
TPAD = T + 8


def _lstm_chunk_kernel(x_ref, mask_ref, hc0_ref, wih0_ref, whh0_ref,
                       wih1_ref, whh1_ref, b0_ref, b1_ref,
                       ys_ref, fin_ref, g0_ref, ma_ref, mb_ref):
    i = pl.program_id(0)
    base = i * TT

    def mrow(idx):
        return mask_ref[pl.ds(idx, 1)].reshape(N, 128)[:, :1]

    @pl.when(i == 0)
    def _():
        fin_ref[...] = hc0_ref[...]
        m0 = mrow(0)
        h0m0 = (hc0_ref[0] * m0).astype(jnp.bfloat16)
        h1m0 = (hc0_ref[1] * m0).astype(jnp.bfloat16)
        ma_ref[...] = jnp.dot(h0m0, whh0_ref[...],
                              preferred_element_type=jnp.float32)
        mb_ref[...] = jnp.dot(h1m0, whh1_ref[...],
                              preferred_element_type=jnp.float32) + b1_ref[...]

    xv = x_ref[...].reshape(TT * N, D).astype(jnp.bfloat16)
    g0_ref[...] = (
        jnp.dot(xv, wih0_ref[...], preferred_element_type=jnp.float32)
        + b0_ref[...]
    )

    init = (fin_ref[0], fin_ref[2], fin_ref[1], fin_ref[3])

    def step(t, carry):
        h0, c0, h1, c1 = carry
        mt = mrow(base + t)
        mt1 = mrow(base + t + 1)

        c0m = c0 * mt
        g0 = g0_ref[pl.ds(t * N, N), :] + ma_ref[...]
        i0 = jax.nn.sigmoid(g0[:, :H])
        f0 = jax.nn.sigmoid(g0[:, H:2 * H])
        gg0 = jnp.tanh(g0[:, 2 * H:3 * H])
        o0 = jax.nn.sigmoid(g0[:, 3 * H:])
        c0n = f0 * c0m + i0 * gg0
        h0n = o0 * jnp.tanh(c0n)
        h0nb = h0n.astype(jnp.bfloat16)

        g1 = jnp.dot(h0nb, wih1_ref[...],
                     preferred_element_type=jnp.float32) + mb_ref[...]

        h0m_next = (h0n * mt1).astype(jnp.bfloat16)
        ma_ref[...] = jnp.dot(h0m_next, whh0_ref[...],
                              preferred_element_type=jnp.float32)

        c1m = c1 * mt
        i1 = jax.nn.sigmoid(g1[:, :H])
        f1 = jax.nn.sigmoid(g1[:, H:2 * H])
        gg1 = jnp.tanh(g1[:, 2 * H:3 * H])
        o1 = jax.nn.sigmoid(g1[:, 3 * H:])
        c1n = f1 * c1m + i1 * gg1
        h1n = o1 * jnp.tanh(c1n)

        h1m_next = (h1n * mt1).astype(jnp.bfloat16)
        mb_ref[...] = jnp.dot(h1m_next, whh1_ref[...],
                              preferred_element_type=jnp.float32) + b1_ref[...]

        ys_ref[pl.ds(t, 1)] = h1n[None]
        return (h0n, c0n, h1n, c1n)

    h0, c0, h1, c1 = jax.lax.fori_loop(0, TT, step, init)
    fin_ref[0] = h0
    fin_ref[1] = h1
    fin_ref[2] = c0
    fin_ref[3] = c1


def kernel(x, rnn_states, dones, W_ih0, W_hh0, b_ih0, b_hh0,
           W_ih1, W_hh1, b_ih1, b_hh1):
    xs = x.reshape(T, N, D)
    mask_b = jnp.zeros((TPAD, N, 128), jnp.float32)
    mask_b = mask_b.at[:T].set(
        jnp.broadcast_to((1.0 - dones.astype(jnp.float32))[:, :, None],
                         (T, N, 128)))
    wih0 = W_ih0.T.astype(jnp.bfloat16)
    whh0 = W_hh0.T.astype(jnp.bfloat16)
    wih1 = W_ih1.T.astype(jnp.bfloat16)
    whh1 = W_hh1.T.astype(jnp.bfloat16)
    b0 = (b_ih0 + b_hh0).reshape(1, 4 * H)
    b1 = (b_ih1 + b_hh1).reshape(1, 4 * H)

    full = lambda shape: pl.BlockSpec(shape, lambda i: (0,) * len(shape))

    ys, fin = pl.pallas_call(
        _lstm_chunk_kernel,
        grid=(GRID,),
        in_specs=[
            pl.BlockSpec((TT, N, D), lambda i: (i, 0, 0)),
            full((TPAD, N, 128)),
            full((2 * L, N, H)),
            full((D, 4 * H)),
            full((H, 4 * H)),
            full((H, 4 * H)),
            full((H, 4 * H)),
            full((1, 4 * H)),
            full((1, 4 * H)),
        ],
        out_specs=[
            pl.BlockSpec((TT, N, H), lambda i: (i, 0, 0)),
            full((2 * L, N, H)),
        ],
        out_shape=[
            jax.ShapeDtypeStruct((T, N, H), jnp.float32),
            jax.ShapeDtypeStruct((2 * L, N, H), jnp.float32),
        ],
        scratch_shapes=[
            pltpu.VMEM((TT * N, 4 * H), jnp.float32),
            pltpu.VMEM((N, 4 * H), jnp.float32),
            pltpu.VMEM((N, 4 * H), jnp.float32),
        ],
    )(xs, mask_b, rnn_states, wih0, whh0, wih1, whh1, b0, b1)

    return ys.reshape(T * N, H), fin


# speedup vs baseline: 1.4391x; 1.4218x over previous
import jax
import jax.numpy as jnp
from jax.experimental import pallas as pl
from jax.experimental.pallas import tpu as pltpu

T, N, D, H, L = 512, 16, 512, 512, 2
TT = 64
GRID = T // TT
TPAD = T + 8


def _lstm_chunk_kernel(x_ref, mask_ref, hc0_ref, wih0_ref, whh0_ref,
                       wih1_ref, whh1_ref, b0_ref, b1_ref,
                       ys_ref, fin_ref, g0_ref, g1p_ref, h0s_ref,
                       ma_ref, mb_ref):
    i = pl.program_id(0)
    base0 = jnp.minimum(i, GRID - 1) * TT
    base1 = jnp.maximum(i - 1, 0) * TT

    def mrow(idx):
        return mask_ref[pl.ds(idx, 1)].reshape(N, 128)[:, :1]

    @pl.when(i == 0)
    def _():
        fin_ref[...] = hc0_ref[...]
        h0m0 = (hc0_ref[0] * mrow(0)).astype(jnp.bfloat16)
        ma_ref[...] = jnp.dot(h0m0, whh0_ref[...],
                              preferred_element_type=jnp.float32)

    @pl.when(i == 1)
    def _():
        h1m0 = (fin_ref[1] * mrow(0)).astype(jnp.bfloat16)
        mb_ref[...] = jnp.dot(h1m0, whh1_ref[...],
                              preferred_element_type=jnp.float32)

    g1p_ref[...] = (
        jnp.dot(h0s_ref[...], wih1_ref[...],
                preferred_element_type=jnp.float32)
        + b1_ref[...]
    )

    xv = x_ref[...].reshape(TT * N, D).astype(jnp.bfloat16)
    g0_ref[...] = (
        jnp.dot(xv, wih0_ref[...], preferred_element_type=jnp.float32)
        + b0_ref[...]
    )

    init = (fin_ref[0], fin_ref[2], fin_ref[1], fin_ref[3])

    def step(t, carry):
        h0, c0, h1, c1 = carry

        mt0 = mrow(base0 + t)
        c0m = c0 * mt0
        g0 = g0_ref[pl.ds(t * N, N), :] + ma_ref[...]
        i0 = jax.nn.sigmoid(g0[:, :H])
        f0 = jax.nn.sigmoid(g0[:, H:2 * H])
        gg0 = jnp.tanh(g0[:, 2 * H:3 * H])
        o0 = jax.nn.sigmoid(g0[:, 3 * H:])
        c0n = f0 * c0m + i0 * gg0
        h0n = o0 * jnp.tanh(c0n)
        h0s_ref[pl.ds(t * N, N), :] = h0n.astype(jnp.bfloat16)

        h0m_next = (h0n * mrow(base0 + t + 1)).astype(jnp.bfloat16)
        ma_ref[...] = jnp.dot(h0m_next, whh0_ref[...],
                              preferred_element_type=jnp.float32)

        mt1 = mrow(base1 + t)
        c1m = c1 * mt1
        g1 = g1p_ref[pl.ds(t * N, N), :] + mb_ref[...]
        i1 = jax.nn.sigmoid(g1[:, :H])
        f1 = jax.nn.sigmoid(g1[:, H:2 * H])
        gg1 = jnp.tanh(g1[:, 2 * H:3 * H])
        o1 = jax.nn.sigmoid(g1[:, 3 * H:])
        c1n = f1 * c1m + i1 * gg1
        h1n = o1 * jnp.tanh(c1n)

        h1m_next = (h1n * mrow(base1 + t + 1)).astype(jnp.bfloat16)
        mb_ref[...] = jnp.dot(h1m_next, whh1_ref[...],
                              preferred_element_type=jnp.float32)

        ys_ref[pl.ds(t, 1)] = h1n[None]
        return (h0n, c0n, h1n, c1n)

    h0, c0, h1, c1 = jax.lax.fori_loop(0, TT, step, init, unroll=16)

    @pl.when(i < GRID)
    def _():
        fin_ref[0] = h0
        fin_ref[2] = c0

    @pl.when(i > 0)
    def _():
        fin_ref[1] = h1
        fin_ref[3] = c1


def kernel(x, rnn_states, dones, W_ih0, W_hh0, b_ih0, b_hh0,
           W_ih1, W_hh1, b_ih1, b_hh1):
    xs = x.reshape(T, N, D)
    mask_b = jnp.zeros((TPAD, N, 128), jnp.float32)
    mask_b = mask_b.at[:T].set(
        jnp.broadcast_to((1.0 - dones.astype(jnp.float32))[:, :, None],
                         (T, N, 128)))
    wih0 = W_ih0.T.astype(jnp.bfloat16)
    whh0 = W_hh0.T.astype(jnp.bfloat16)
    wih1 = W_ih1.T.astype(jnp.bfloat16)
    whh1 = W_hh1.T.astype(jnp.bfloat16)
    b0 = (b_ih0 + b_hh0).reshape(1, 4 * H)
    b1 = (b_ih1 + b_hh1).reshape(1, 4 * H)

    full = lambda shape: pl.BlockSpec(shape, lambda i: (0,) * len(shape))

    ys, fin = pl.pallas_call(
        _lstm_chunk_kernel,
        grid=(GRID + 1,),
        in_specs=[
            pl.BlockSpec((TT, N, D),
                         lambda i: (jnp.minimum(i, GRID - 1), 0, 0)),
            full((TPAD, N, 128)),
            full((2 * L, N, H)),
            full((D, 4 * H)),
            full((H, 4 * H)),
            full((H, 4 * H)),
            full((H, 4 * H)),
            full((1, 4 * H)),
            full((1, 4 * H)),
        ],
        out_specs=[
            pl.BlockSpec((TT, N, H),
                         lambda i: (jnp.maximum(i - 1, 0), 0, 0)),
            full((2 * L, N, H)),
        ],
        out_shape=[
            jax.ShapeDtypeStruct((T, N, H), jnp.float32),
            jax.ShapeDtypeStruct((2 * L, N, H), jnp.float32),
        ],
        scratch_shapes=[
            pltpu.VMEM((TT * N, 4 * H), jnp.float32),
            pltpu.VMEM((TT * N, 4 * H), jnp.float32),
            pltpu.VMEM((TT * N, H), jnp.bfloat16),
            pltpu.VMEM((N, 4 * H), jnp.float32),
            pltpu.VMEM((N, 4 * H), jnp.float32),
        ],
    )(xs, mask_b, rnn_states, wih0, whh0, wih1, whh1, b0, b1)

    return ys.reshape(T * N, H), fin


# speedup vs baseline: 1.5063x; 1.0467x over previous
import jax
import jax.numpy as jnp
from jax.experimental import pallas as pl
from jax.experimental.pallas import tpu as pltpu

T, N, D, H, L = 512, 16, 512, 512, 2
TT = 32
GRID = T // TT
TPAD = T + 8


def _lstm_chunk_kernel(x_ref, mask_ref, hc0_ref, wih0_ref, whh0_ref,
                       wih1_ref, whh1_ref, b0_ref, b1_ref,
                       ys_ref, fin_ref, g0_ref, g1p_ref, h0s_ref,
                       ma_ref, mb_ref):
    i = pl.program_id(0)
    base0 = jnp.minimum(i, GRID - 1) * TT
    base1 = jnp.maximum(i - 1, 0) * TT

    def mrow(idx):
        return mask_ref[pl.ds(idx, 1)].reshape(N, 128)[:, :1]

    @pl.when(i == 0)
    def _():
        fin_ref[...] = hc0_ref[...]
        h0m0 = (hc0_ref[0] * mrow(0)).astype(jnp.bfloat16)
        ma_ref[...] = jnp.dot(h0m0, whh0_ref[...],
                              preferred_element_type=jnp.float32)

    @pl.when(i == 1)
    def _():
        h1m0 = (fin_ref[1] * mrow(0)).astype(jnp.bfloat16)
        mb_ref[...] = jnp.dot(h1m0, whh1_ref[...],
                              preferred_element_type=jnp.float32)

    g1p_ref[...] = (
        jnp.dot(h0s_ref[...], wih1_ref[...],
                preferred_element_type=jnp.float32)
        + b1_ref[...]
    )

    xv = x_ref[...].reshape(TT * N, D).astype(jnp.bfloat16)
    g0_ref[...] = (
        jnp.dot(xv, wih0_ref[...], preferred_element_type=jnp.float32)
        + b0_ref[...]
    )

    init = (fin_ref[0], fin_ref[2], fin_ref[1], fin_ref[3])

    def step(t, carry):
        h0, c0, h1, c1 = carry

        mt0 = mrow(base0 + t)
        c0m = c0 * mt0
        g0 = g0_ref[pl.ds(t * N, N), :] + ma_ref[...]
        i0 = jax.nn.sigmoid(g0[:, :H])
        f0 = jax.nn.sigmoid(g0[:, H:2 * H])
        gg0 = jnp.tanh(g0[:, 2 * H:3 * H])
        o0 = jax.nn.sigmoid(g0[:, 3 * H:])
        c0n = f0 * c0m + i0 * gg0
        h0n = o0 * jnp.tanh(c0n)
        h0s_ref[pl.ds(t * N, N), :] = h0n.astype(jnp.bfloat16)

        h0m_next = (h0n * mrow(base0 + t + 1)).astype(jnp.bfloat16)
        ma_ref[...] = jnp.dot(h0m_next, whh0_ref[...],
                              preferred_element_type=jnp.float32)

        mt1 = mrow(base1 + t)
        c1m = c1 * mt1
        g1 = g1p_ref[pl.ds(t * N, N), :] + mb_ref[...]
        i1 = jax.nn.sigmoid(g1[:, :H])
        f1 = jax.nn.sigmoid(g1[:, H:2 * H])
        gg1 = jnp.tanh(g1[:, 2 * H:3 * H])
        o1 = jax.nn.sigmoid(g1[:, 3 * H:])
        c1n = f1 * c1m + i1 * gg1
        h1n = o1 * jnp.tanh(c1n)

        h1m_next = (h1n * mrow(base1 + t + 1)).astype(jnp.bfloat16)
        mb_ref[...] = jnp.dot(h1m_next, whh1_ref[...],
                              preferred_element_type=jnp.float32)

        ys_ref[pl.ds(t, 1)] = h1n[None]
        return (h0n, c0n, h1n, c1n)

    h0, c0, h1, c1 = jax.lax.fori_loop(0, TT, step, init, unroll=16)

    @pl.when(i < GRID)
    def _():
        fin_ref[0] = h0
        fin_ref[2] = c0

    @pl.when(i > 0)
    def _():
        fin_ref[1] = h1
        fin_ref[3] = c1


def kernel(x, rnn_states, dones, W_ih0, W_hh0, b_ih0, b_hh0,
           W_ih1, W_hh1, b_ih1, b_hh1):
    xs = x.reshape(T, N, D)
    mask_b = jnp.zeros((TPAD, N, 128), jnp.float32)
    mask_b = mask_b.at[:T].set(
        jnp.broadcast_to((1.0 - dones.astype(jnp.float32))[:, :, None],
                         (T, N, 128)))
    wih0 = W_ih0.T.astype(jnp.bfloat16)
    whh0 = W_hh0.T.astype(jnp.bfloat16)
    wih1 = W_ih1.T.astype(jnp.bfloat16)
    whh1 = W_hh1.T.astype(jnp.bfloat16)
    b0 = (b_ih0 + b_hh0).reshape(1, 4 * H)
    b1 = (b_ih1 + b_hh1).reshape(1, 4 * H)

    full = lambda shape: pl.BlockSpec(shape, lambda i: (0,) * len(shape))

    ys, fin = pl.pallas_call(
        _lstm_chunk_kernel,
        grid=(GRID + 1,),
        in_specs=[
            pl.BlockSpec((TT, N, D),
                         lambda i: (jnp.minimum(i, GRID - 1), 0, 0)),
            full((TPAD, N, 128)),
            full((2 * L, N, H)),
            full((D, 4 * H)),
            full((H, 4 * H)),
            full((H, 4 * H)),
            full((H, 4 * H)),
            full((1, 4 * H)),
            full((1, 4 * H)),
        ],
        out_specs=[
            pl.BlockSpec((TT, N, H),
                         lambda i: (jnp.maximum(i - 1, 0), 0, 0)),
            full((2 * L, N, H)),
        ],
        out_shape=[
            jax.ShapeDtypeStruct((T, N, H), jnp.float32),
            jax.ShapeDtypeStruct((2 * L, N, H), jnp.float32),
        ],
        scratch_shapes=[
            pltpu.VMEM((TT * N, 4 * H), jnp.float32),
            pltpu.VMEM((TT * N, 4 * H), jnp.float32),
            pltpu.VMEM((TT * N, H), jnp.bfloat16),
            pltpu.VMEM((N, 4 * H), jnp.float32),
            pltpu.VMEM((N, 4 * H), jnp.float32),
        ],
    )(xs, mask_b, rnn_states, wih0, whh0, wih1, whh1, b0, b1)

    return ys.reshape(T * N, H), fin


# speedup vs baseline: 1.5523x; 1.0305x over previous
import jax
import jax.numpy as jnp
from jax.experimental import pallas as pl
from jax.experimental.pallas import tpu as pltpu

T, N, D, H, L = 512, 16, 512, 512, 2
TT = 16
GRID = T // TT
TPAD = T + 8


def _lstm_chunk_kernel(x_ref, mask_ref, hc0_ref, wih0_ref, whh0_ref,
                       wih1_ref, whh1_ref, b0_ref, b1_ref,
                       ys_ref, fin_ref, g0_ref, g1p_ref, h0s_ref,
                       ma_ref, mb_ref):
    i = pl.program_id(0)
    base0 = jnp.minimum(i, GRID - 1) * TT
    base1 = jnp.maximum(i - 1, 0) * TT

    def mrow(idx):
        return mask_ref[pl.ds(idx, 1)].reshape(N, 128)[:, :1]

    @pl.when(i == 0)
    def _():
        fin_ref[...] = hc0_ref[...]
        h0m0 = (hc0_ref[0] * mrow(0)).astype(jnp.bfloat16)
        ma_ref[...] = jnp.dot(h0m0, whh0_ref[...],
                              preferred_element_type=jnp.float32)

    @pl.when(i == 1)
    def _():
        h1m0 = (fin_ref[1] * mrow(0)).astype(jnp.bfloat16)
        mb_ref[...] = jnp.dot(h1m0, whh1_ref[...],
                              preferred_element_type=jnp.float32)

    g1p_ref[...] = (
        jnp.dot(h0s_ref[...], wih1_ref[...],
                preferred_element_type=jnp.float32)
        + b1_ref[...]
    )

    xv = x_ref[...].reshape(TT * N, D).astype(jnp.bfloat16)
    g0_ref[...] = (
        jnp.dot(xv, wih0_ref[...], preferred_element_type=jnp.float32)
        + b0_ref[...]
    )

    init = (fin_ref[0], fin_ref[2], fin_ref[1], fin_ref[3])

    def step(t, carry):
        h0, c0, h1, c1 = carry

        mt0 = mrow(base0 + t)
        c0m = c0 * mt0
        g0 = g0_ref[pl.ds(t * N, N), :] + ma_ref[...]
        i0 = jax.nn.sigmoid(g0[:, :H])
        f0 = jax.nn.sigmoid(g0[:, H:2 * H])
        gg0 = jnp.tanh(g0[:, 2 * H:3 * H])
        o0 = jax.nn.sigmoid(g0[:, 3 * H:])
        c0n = f0 * c0m + i0 * gg0
        h0n = o0 * jnp.tanh(c0n)
        h0s_ref[pl.ds(t * N, N), :] = h0n.astype(jnp.bfloat16)

        h0m_next = (h0n * mrow(base0 + t + 1)).astype(jnp.bfloat16)
        ma_ref[...] = jnp.dot(h0m_next, whh0_ref[...],
                              preferred_element_type=jnp.float32)

        mt1 = mrow(base1 + t)
        c1m = c1 * mt1
        g1 = g1p_ref[pl.ds(t * N, N), :] + mb_ref[...]
        i1 = jax.nn.sigmoid(g1[:, :H])
        f1 = jax.nn.sigmoid(g1[:, H:2 * H])
        gg1 = jnp.tanh(g1[:, 2 * H:3 * H])
        o1 = jax.nn.sigmoid(g1[:, 3 * H:])
        c1n = f1 * c1m + i1 * gg1
        h1n = o1 * jnp.tanh(c1n)

        h1m_next = (h1n * mrow(base1 + t + 1)).astype(jnp.bfloat16)
        mb_ref[...] = jnp.dot(h1m_next, whh1_ref[...],
                              preferred_element_type=jnp.float32)

        ys_ref[pl.ds(t, 1)] = h1n[None]
        return (h0n, c0n, h1n, c1n)

    h0, c0, h1, c1 = jax.lax.fori_loop(0, TT, step, init, unroll=16)

    @pl.when(i < GRID)
    def _():
        fin_ref[0] = h0
        fin_ref[2] = c0

    @pl.when(i > 0)
    def _():
        fin_ref[1] = h1
        fin_ref[3] = c1


def kernel(x, rnn_states, dones, W_ih0, W_hh0, b_ih0, b_hh0,
           W_ih1, W_hh1, b_ih1, b_hh1):
    xs = x.reshape(T, N, D)
    mask_b = jnp.zeros((TPAD, N, 128), jnp.float32)
    mask_b = mask_b.at[:T].set(
        jnp.broadcast_to((1.0 - dones.astype(jnp.float32))[:, :, None],
                         (T, N, 128)))
    wih0 = W_ih0.T.astype(jnp.bfloat16)
    whh0 = W_hh0.T.astype(jnp.bfloat16)
    wih1 = W_ih1.T.astype(jnp.bfloat16)
    whh1 = W_hh1.T.astype(jnp.bfloat16)
    b0 = (b_ih0 + b_hh0).reshape(1, 4 * H)
    b1 = (b_ih1 + b_hh1).reshape(1, 4 * H)

    full = lambda shape: pl.BlockSpec(shape, lambda i: (0,) * len(shape))

    ys, fin = pl.pallas_call(
        _lstm_chunk_kernel,
        grid=(GRID + 1,),
        in_specs=[
            pl.BlockSpec((TT, N, D),
                         lambda i: (jnp.minimum(i, GRID - 1), 0, 0)),
            full((TPAD, N, 128)),
            full((2 * L, N, H)),
            full((D, 4 * H)),
            full((H, 4 * H)),
            full((H, 4 * H)),
            full((H, 4 * H)),
            full((1, 4 * H)),
            full((1, 4 * H)),
        ],
        out_specs=[
            pl.BlockSpec((TT, N, H),
                         lambda i: (jnp.maximum(i - 1, 0), 0, 0)),
            full((2 * L, N, H)),
        ],
        out_shape=[
            jax.ShapeDtypeStruct((T, N, H), jnp.float32),
            jax.ShapeDtypeStruct((2 * L, N, H), jnp.float32),
        ],
        scratch_shapes=[
            pltpu.VMEM((TT * N, 4 * H), jnp.float32),
            pltpu.VMEM((TT * N, 4 * H), jnp.float32),
            pltpu.VMEM((TT * N, H), jnp.bfloat16),
            pltpu.VMEM((N, 4 * H), jnp.float32),
            pltpu.VMEM((N, 4 * H), jnp.float32),
        ],
    )(xs, mask_b, rnn_states, wih0, whh0, wih1, whh1, b0, b1)

    return ys.reshape(T * N, H), fin
